# baseline (speedup 1.0000x reference)
import functools
import math

import jax
import jax.numpy as jnp
import numpy as np
from jax import lax
from jax.experimental import pallas as pl
from jax.experimental.pallas import tpu as pltpu
from jax.experimental.pallas import tpu_sc as plsc

B = 4096
L = 50
C = 64
SCALE = math.sqrt(C)

NC = 2
NS = 16
NW = NC * NS

B_PER_W = B // NW
G = 8
CHUNKS = B_PER_W // G
IDX_PER_CHUNK = G * L
GATHER_BATCH = 100
NGB = IDX_PER_CHUNK // GATHER_BATCH
OUT_PER_CHUNK = G * C * L
VECS_PER_B = (C * L) // 16

_f = np.arange(C * L, dtype=np.int32)
PERM_L = _f % L
PERM_C = _f // L


@functools.partial(
    pl.kernel,
    out_type=jax.ShapeDtypeStruct((B * C * L,), jnp.float32),
    mesh=plsc.VectorSubcoreMesh(core_axis_name="c", subcore_axis_name="s"),
    scratch_types=[
        pltpu.VMEM((NGB, GATHER_BATCH), jnp.int32),
        pltpu.VMEM((IDX_PER_CHUNK, C), jnp.float32),
        pltpu.VMEM((OUT_PER_CHUNK,), jnp.float32),
        pltpu.VMEM((C * L,), jnp.int32),
        pltpu.VMEM((C * L,), jnp.int32),
        pltpu.SemaphoreType.DMA,
    ],
    compiler_params=pltpu.CompilerParams(
        use_tc_tiling_on_sc=False, needs_layout_passes=False),
)
def _emb_lookup(x_hbm, perml_hbm, permc_hbm, table_hbm, out_hbm,
                idx_v, rows_v, outbuf, perml_v, permc_v, sem):
    wid = lax.axis_index("s") * NC + lax.axis_index("c")
    pltpu.sync_copy(perml_hbm, perml_v)
    pltpu.sync_copy(permc_hbm, permc_v)
    xrow0 = wid * (B_PER_W * L // GATHER_BATCH)
    obase = wid * (B_PER_W * C * L)

    def chunk_body(k, _):
        pltpu.sync_copy(x_hbm.at[pl.ds(xrow0 + k * NGB, NGB)], idx_v)
        copies = []
        for j in range(NGB):
            copies.append(pltpu.async_copy(
                table_hbm.at[idx_v.at[j]],
                rows_v.at[pl.ds(j * GATHER_BATCH, GATHER_BATCH)],
                sem))
        for cp in copies:
            cp.wait()
        for bofs in range(G):
            rbase = bofs * L
            wbase = bofs * C * L

            def vbody(v, _, rbase=rbase, wbase=wbase):
                s = v * 16
                lvec = perml_v[pl.ds(s, 16)] + rbase
                cvec = permc_v[pl.ds(s, 16)]
                vals = plsc.load_gather(rows_v, [lvec, cvec]) * SCALE
                outbuf[pl.ds(wbase + s, 16)] = vals
                return 0

            lax.fori_loop(0, VECS_PER_B, vbody, 0)
        pltpu.sync_copy(outbuf,
                        out_hbm.at[pl.ds(obase + k * OUT_PER_CHUNK,
                                         OUT_PER_CHUNK)])
        return 0

    lax.fori_loop(0, CHUNKS, chunk_body, 0)


def kernel(x, emb_weight):
    x2 = x.reshape(B * L // GATHER_BATCH, GATHER_BATCH).astype(jnp.int32)
    out_flat = _emb_lookup(x2, jnp.asarray(PERM_L), jnp.asarray(PERM_C),
                           emb_weight)
    return out_flat.reshape(B, C, L)
